# baseline (speedup 1.0000x reference)
import jax
import jax.numpy as jnp
from jax.experimental import pallas as pl
from jax.experimental.pallas import tpu as pltpu

_B = 4
_N = 256
_E1 = (_N * (_N - 1)) // 2


def _edge_index_body(out_ref):
    boff = (pl.program_id(0) * _N).astype(jnp.float32)
    ef = jax.lax.broadcasted_iota(jnp.int32, (2, _E1), 1).astype(jnp.float32)
    s = jnp.sqrt((8.0 * _E1 - 6.5) - 8.0 * ef)
    t = jnp.floor(0.5 * s - 0.5)
    rowf = (boff + (_N - 2.0)) - t
    colf = t * (0.5 * t + 0.5) + (ef + (boff + (_N - _E1)))
    m = jax.lax.broadcasted_iota(jnp.int32, (2, 1), 0) == 0
    out_ref[:, :] = jnp.where(m, rowf, colf).astype(jnp.int32)


def kernel(x_topology, x_temporal, W_gnn, b_gnn, W_mean, b_mean, W_var, b_var, W_w, b_w):
    return pl.pallas_call(
        _edge_index_body,
        grid=(_B,),
        out_specs=pl.BlockSpec((2, _E1), lambda k: (0, k)),
        out_shape=jax.ShapeDtypeStruct((2, _B * _E1), jnp.int32),
        compiler_params=pltpu.CompilerParams(dimension_semantics=("arbitrary",)),
    )()


# speedup vs baseline: 1.9469x; 1.9469x over previous
import jax
import jax.numpy as jnp
from jax.experimental import pallas as pl

_B = 4
_N = 256
_E1 = (_N * (_N - 1)) // 2


def _edge_index_body(out_ref):
    ef = jax.lax.broadcasted_iota(jnp.int32, (2, _E1), 1).astype(jnp.float32)
    s = jnp.sqrt((8.0 * _E1 - 6.5) - 8.0 * ef)
    t = jnp.floor(0.5 * s - 0.5)
    rowf = (_N - 2.0) - t
    colf = t * (0.5 * t + 0.5) + (ef + (_N - _E1))
    m = jax.lax.broadcasted_iota(jnp.int32, (2, 1), 0) == 0
    v = jnp.where(m, rowf, colf).astype(jnp.int32)
    for k in range(_B):
        out_ref[:, k * _E1:(k + 1) * _E1] = v + (k * _N)


def kernel(x_topology, x_temporal, W_gnn, b_gnn, W_mean, b_mean, W_var, b_var, W_w, b_w):
    return pl.pallas_call(
        _edge_index_body,
        out_shape=jax.ShapeDtypeStruct((2, _B * _E1), jnp.int32),
    )()
